# Initial kernel scaffold; baseline (speedup 1.0000x reference)
import jax
import jax.numpy as jnp
from jax.experimental import pallas as pl


def kernel(input_ids, word_emb, pos_emb, ln_gamma, ln_beta):
    raise NotImplementedError("write your pallas kernel here")


# speedup vs baseline: 1.5818x; 1.5818x over previous
import functools

import jax
import jax.numpy as jnp
from jax import lax
from jax.experimental import pallas as pl
from jax.experimental.pallas import tpu as pltpu
from jax.experimental.pallas import tpu_sc as plsc

_B = 4096
_S = 200
_H = 64
_R = _B * _S
_NW = 32
_RPW = _R // _NW
_IDS_MINOR = 128
_CHUNK = 1024
_NIDX = _CHUNK // _IDS_MINOR
_NCHUNK = _RPW // _CHUNK
_EPS = 1e-12
_MAGIC = 0x5F3759DF


_GATHER_DNUMS = lax.GatherDimensionNumbers(
    offset_dims=(), collapsed_slice_dims=(0,), start_index_map=(0,))


def _allsum16(v, perms):
    for p in perms:
        v = v + lax.gather(v, p, _GATHER_DNUMS, (1,),
                           mode=lax.GatherScatterMode.PROMISE_IN_BOUNDS)
    return v


def _rsqrt16(a):
    ai = lax.bitcast_convert_type(a, jnp.int32)
    yi = jnp.int32(_MAGIC) - (ai >> 1)
    y = lax.bitcast_convert_type(yi, jnp.float32)
    ha = a * jnp.float32(0.5)
    for _ in range(3):
        y = y * (jnp.float32(1.5) - ha * y * y)
    return y


def _body(ids_hbm, wemb_hbm, pos_hbm, gam_hbm, bet_hbm, out_hbm,
          idx_v, rows_v, pos_v, g_v, b_v, sem):
    wid = lax.axis_index("s") * 2 + lax.axis_index("c")

    pltpu.sync_copy(pos_hbm, pos_v)
    pltpu.sync_copy(gam_hbm, g_v)
    pltpu.sync_copy(bet_hbm, b_v)
    gk = [g_v[pl.ds(k * 16, 16)] for k in range(4)]
    bk = [b_v[pl.ds(k * 16, 16)] for k in range(4)]

    inv_h = jnp.float32(1.0 / _H)
    perms = [(lax.iota(jnp.int32, 16) ^ jnp.int32(k))[:, None]
             for k in (1, 2, 4, 8)]

    def chunk_body(c, carry):
        base = pl.multiple_of(wid * _RPW + c * _CHUNK, _CHUNK)
        ib = pl.multiple_of(base // _IDS_MINOR, _NIDX)
        pltpu.sync_copy(ids_hbm.at[pl.ds(ib, _NIDX)], idx_v)
        cps = [
            pltpu.async_copy(
                wemb_hbm.at[idx_v.at[j]],
                rows_v.at[pl.ds(j * _IDS_MINOR, _IDS_MINOR)], sem)
            for j in range(_NIDX)
        ]
        for cp in cps:
            cp.wait()

        def row_body(t, rcarry):
            p = lax.rem(c * _CHUNK + t, _S)
            x = [rows_v[t, pl.ds(k * 16, 16)] + pos_v[p, pl.ds(k * 16, 16)]
                 for k in range(4)]
            s = (x[0] + x[1]) + (x[2] + x[3])
            q = (x[0] * x[0] + x[1] * x[1]) + (x[2] * x[2] + x[3] * x[3])
            meanv = _allsum16(s, perms) * inv_h
            varv = _allsum16(q, perms) * inv_h - meanv * meanv
            rstd = _rsqrt16(varv + jnp.float32(_EPS))
            for k in range(4):
                y = (x[k] - meanv) * rstd * gk[k] + bk[k]
                rows_v[t, pl.ds(k * 16, 16)] = y
            return rcarry

        lax.fori_loop(0, _CHUNK, row_body, 0, unroll=False)
        pltpu.sync_copy(rows_v, out_hbm.at[pl.ds(base, _CHUNK)])
        return carry

    lax.fori_loop(0, _NCHUNK, chunk_body, 0, unroll=False)


_emb_ln = functools.partial(
    pl.kernel,
    mesh=plsc.VectorSubcoreMesh(core_axis_name="c", subcore_axis_name="s"),
    compiler_params=pltpu.CompilerParams(use_tc_tiling_on_sc=False),
    out_type=jax.ShapeDtypeStruct((_R, _H), jnp.float32),
    scratch_types=[
        pltpu.VMEM((_NIDX, _IDS_MINOR), jnp.int32),
        pltpu.VMEM((_CHUNK, _H), jnp.float32),
        pltpu.VMEM((_S, _H), jnp.float32),
        pltpu.VMEM((_H,), jnp.float32),
        pltpu.VMEM((_H,), jnp.float32),
        pltpu.SemaphoreType.DMA,
    ],
)(_body)


def kernel(input_ids, word_emb, pos_emb, ln_gamma, ln_beta):
    ids2 = input_ids.reshape(_R // _IDS_MINOR, _IDS_MINOR).astype(jnp.int32)
    out = _emb_ln(ids2, word_emb, pos_emb, ln_gamma, ln_beta)
    return out.reshape(_B, _S, _H)


# speedup vs baseline: 1.5945x; 1.0080x over previous
import functools

import jax
import jax.numpy as jnp
from jax import lax
from jax.experimental import pallas as pl
from jax.experimental.pallas import tpu as pltpu
from jax.experimental.pallas import tpu_sc as plsc

_B = 4096
_S = 200
_H = 64
_R = _B * _S
_NW = 32
_RPW = _R // _NW
_IDS_MINOR = 128
_CHUNK = 1024
_NIDX = _CHUNK // _IDS_MINOR
_NCHUNK = _RPW // _CHUNK
_EPS = 1e-12
_MAGIC = 0x5F3759DF


_GATHER_DNUMS = lax.GatherDimensionNumbers(
    offset_dims=(), collapsed_slice_dims=(0,), start_index_map=(0,))


def _allsum16(v, perms):
    for p in perms:
        v = v + lax.gather(v, p, _GATHER_DNUMS, (1,),
                           mode=lax.GatherScatterMode.PROMISE_IN_BOUNDS)
    return v


def _rsqrt16(a):
    ai = lax.bitcast_convert_type(a, jnp.int32)
    yi = jnp.int32(_MAGIC) - (ai >> 1)
    y = lax.bitcast_convert_type(yi, jnp.float32)
    ha = a * jnp.float32(0.5)
    for _ in range(3):
        y = y * (jnp.float32(1.5) - ha * y * y)
    return y


def _body(ids_hbm, wemb_hbm, pos_hbm, gam_hbm, bet_hbm, out_hbm,
          idx_v, rows_v, pos_v, g_v, b_v, sem):
    wid = lax.axis_index("s") * 2 + lax.axis_index("c")

    pltpu.sync_copy(pos_hbm, pos_v)
    pltpu.sync_copy(gam_hbm, g_v)
    pltpu.sync_copy(bet_hbm, b_v)
    gk = [g_v[pl.ds(k * 16, 16)] for k in range(4)]
    bk = [b_v[pl.ds(k * 16, 16)] for k in range(4)]

    inv_h = jnp.float32(1.0 / _H)
    perms = [(lax.iota(jnp.int32, 16) ^ jnp.int32(k))[:, None]
             for k in (1, 2, 4, 8)]

    def chunk_body(c, carry):
        base = pl.multiple_of(wid * _RPW + c * _CHUNK, _CHUNK)
        ib = pl.multiple_of(base // _IDS_MINOR, _NIDX)
        pltpu.sync_copy(ids_hbm.at[pl.ds(ib, _NIDX)], idx_v)
        cps = [
            pltpu.async_copy(
                wemb_hbm.at[idx_v.at[j]],
                rows_v.at[pl.ds(j * _IDS_MINOR, _IDS_MINOR)], sem)
            for j in range(_NIDX)
        ]
        for cp in cps:
            cp.wait()

        def row_body(t, rcarry):
            p = lax.rem(c * _CHUNK + t, _S)
            x = [rows_v[t, pl.ds(k * 16, 16)] + pos_v[p, pl.ds(k * 16, 16)]
                 for k in range(4)]
            s = (x[0] + x[1]) + (x[2] + x[3])
            q = (x[0] * x[0] + x[1] * x[1]) + (x[2] * x[2] + x[3] * x[3])
            meanv = _allsum16(s, perms) * inv_h
            varv = _allsum16(q, perms) * inv_h - meanv * meanv
            rstd = _rsqrt16(varv + jnp.float32(_EPS))
            for k in range(4):
                y = (x[k] - meanv) * rstd * gk[k] + bk[k]
                rows_v[t, pl.ds(k * 16, 16)] = y
            return rcarry

        lax.fori_loop(0, _CHUNK, row_body, 0, unroll=4)
        pltpu.sync_copy(rows_v, out_hbm.at[pl.ds(base, _CHUNK)])
        return carry

    lax.fori_loop(0, _NCHUNK, chunk_body, 0, unroll=False)


_emb_ln = functools.partial(
    pl.kernel,
    mesh=plsc.VectorSubcoreMesh(core_axis_name="c", subcore_axis_name="s"),
    compiler_params=pltpu.CompilerParams(use_tc_tiling_on_sc=False),
    out_type=jax.ShapeDtypeStruct((_R, _H), jnp.float32),
    scratch_types=[
        pltpu.VMEM((_NIDX, _IDS_MINOR), jnp.int32),
        pltpu.VMEM((_CHUNK, _H), jnp.float32),
        pltpu.VMEM((_S, _H), jnp.float32),
        pltpu.VMEM((_H,), jnp.float32),
        pltpu.VMEM((_H,), jnp.float32),
        pltpu.SemaphoreType.DMA,
    ],
)(_body)


def kernel(input_ids, word_emb, pos_emb, ln_gamma, ln_beta):
    ids2 = input_ids.reshape(_R // _IDS_MINOR, _IDS_MINOR).astype(jnp.int32)
    out = _emb_ln(ids2, word_emb, pos_emb, ln_gamma, ln_beta)
    return out.reshape(_B, _S, _H)
